# baseline (speedup 1.0000x reference)
import functools

import jax
import jax.numpy as jnp
from jax import lax
from jax.experimental import pallas as pl
from jax.experimental.pallas import tpu as pltpu
from jax.experimental.pallas import tpu_sc as plsc

_W = 2
_K = 5
_KK = _K * _K


def kernel(grid_input, w, memory):
    B = grid_input.shape[0]
    N, M, H = memory.shape
    assert B == 4096 and N == 512 and M == 512 and H == 256
    table = memory.reshape(N * M, H)
    grid_adj = grid_input + (w - _W)
    qx = grid_adj[:, 0]
    qy = grid_adj[:, 1]

    R = B * _KK
    NC, NS, L = 2, 16, 16
    NW = NC * NS
    rows_per_w = R // NW
    CHUNK = 128
    n_chunks = rows_per_w // CHUNK

    mesh = plsc.VectorSubcoreMesh(core_axis_name="c", subcore_axis_name="s")

    @functools.partial(
        pl.kernel,
        mesh=mesh,
        out_type=jax.ShapeDtypeStruct((R, H), jnp.float32),
        scratch_types=[
            pltpu.VMEM((B,), jnp.int32),
            pltpu.VMEM((B,), jnp.int32),
            pltpu.VMEM((CHUNK,), jnp.int32),
            pltpu.VMEM((CHUNK, H), jnp.float32),
            pltpu.SemaphoreType.DMA,
        ],
    )
    def gather_rows(qx_hbm, qy_hbm, table_hbm, out_hbm,
                    qx_v, qy_v, idx_v, rows_v, sem):
        wid = lax.axis_index("s") * NC + lax.axis_index("c")
        base = wid * rows_per_w
        pltpu.sync_copy(qx_hbm, qx_v)
        pltpu.sync_copy(qy_hbm, qy_v)

        def body(c, carry):
            start = base + c * CHUNK
            q = lax.shift_right_logical(start, 12)
            i = lax.shift_right_logical(q * 13108, 16)
            j = q - i * _K
            b0 = jnp.bitwise_and(start, B - 1)
            for v in range(CHUNK // L):
                gx = qx_v[pl.ds(b0 + v * L, L)]
                gy = qy_v[pl.ds(b0 + v * L, L)]
                rx = jnp.bitwise_and(gx + (i - _W), N - 1)
                ry = jnp.bitwise_and(gy + (j - _W), M - 1)
                idx_v[pl.ds(v * L, L)] = lax.shift_left(rx, 9) + ry
            pltpu.async_copy(table_hbm.at[idx_v], rows_v, sem).wait()
            pltpu.sync_copy(rows_v, out_hbm.at[pl.ds(start, CHUNK)])
            return carry

        lax.fori_loop(0, n_chunks, body, 0)

    out = gather_rows(qx, qy, table)
    return out.reshape(B, _KK, H)


# speedup vs baseline: 1.0566x; 1.0566x over previous
import functools

import jax
import jax.numpy as jnp
from jax import lax
from jax.experimental import pallas as pl
from jax.experimental.pallas import tpu as pltpu
from jax.experimental.pallas import tpu_sc as plsc

_W = 2
_K = 5
_KK = _K * _K


def kernel(grid_input, w, memory):
    B = grid_input.shape[0]
    N, M, H = memory.shape
    assert B == 4096 and N == 512 and M == 512 and H == 256
    table = memory.reshape(N * M, H)
    grid_adj = grid_input + (w - _W)
    qx = grid_adj[:, 0]
    qy = grid_adj[:, 1]

    R = B * _KK
    NC, NS, L = 2, 16, 16
    NW = NC * NS
    rows_per_w = R // NW
    CHUNK = 128
    n_chunks = rows_per_w // CHUNK

    mesh = plsc.VectorSubcoreMesh(core_axis_name="c", subcore_axis_name="s")

    @functools.partial(
        pl.kernel,
        mesh=mesh,
        out_type=jax.ShapeDtypeStruct((R, H), jnp.float32),
        scratch_types=[
            pltpu.VMEM((B,), jnp.int32),
            pltpu.VMEM((B,), jnp.int32),
            pltpu.VMEM((CHUNK,), jnp.int32),
            pltpu.VMEM((CHUNK,), jnp.int32),
            pltpu.VMEM((CHUNK, H), jnp.float32),
            pltpu.VMEM((CHUNK, H), jnp.float32),
            pltpu.SemaphoreType.DMA,
            pltpu.SemaphoreType.DMA,
            pltpu.SemaphoreType.DMA,
            pltpu.SemaphoreType.DMA,
        ],
    )
    def gather_rows(qx_hbm, qy_hbm, table_hbm, out_hbm,
                    qx_v, qy_v, idx0_v, idx1_v, rows0_v, rows1_v,
                    sg0, sg1, sw0, sw1):
        wid = lax.axis_index("s") * NC + lax.axis_index("c")
        base = wid * rows_per_w
        pltpu.sync_copy(qx_hbm, qx_v)
        pltpu.sync_copy(qy_hbm, qy_v)

        idx_v = (idx0_v, idx1_v)
        rows_v = (rows0_v, rows1_v)
        sg = (sg0, sg1)
        sw = (sw0, sw1)

        def compute_idx(c, dst):
            start = base + c * CHUNK
            q = lax.shift_right_logical(start, 12)
            i = lax.shift_right_logical(q * 13108, 16)
            j = q - i * _K
            b0 = jnp.bitwise_and(start, B - 1)
            for v in range(CHUNK // L):
                gx = qx_v[pl.ds(b0 + v * L, L)]
                gy = qy_v[pl.ds(b0 + v * L, L)]
                rx = jnp.bitwise_and(gx + (i - _W), N - 1)
                ry = jnp.bitwise_and(gy + (j - _W), M - 1)
                dst[pl.ds(v * L, L)] = lax.shift_left(rx, 9) + ry

        def fire_gather(c, p):
            pltpu.async_copy(table_hbm.at[idx_v[p]], rows_v[p], sg[p])

        def wait_gather(p):
            pltpu.make_async_copy(table_hbm.at[idx_v[p]], rows_v[p],
                                  sg[p]).wait()

        def fire_wb(c, p):
            pltpu.async_copy(rows_v[p],
                             out_hbm.at[pl.ds(base + c * CHUNK, CHUNK)], sw[p])

        def wait_wb(c, p):
            pltpu.make_async_copy(rows_v[p],
                                  out_hbm.at[pl.ds(base + c * CHUNK, CHUNK)],
                                  sw[p]).wait()

        compute_idx(0, idx_v[0])
        fire_gather(0, 0)
        compute_idx(1, idx_v[1])
        fire_gather(1, 1)

        def step(c, p):
            wait_gather(p)
            fire_wb(c, p)
            compute_idx(c + 2, idx_v[p])
            wait_wb(c, p)
            fire_gather(c + 2, p)

        def body(it, carry):
            c0 = it * 2
            step(c0, 0)
            step(c0 + 1, 1)
            return carry

        lax.fori_loop(0, 11, body, 0)
        step(22, 0)
        wait_gather(1)
        fire_wb(23, 1)
        wait_gather(0)
        fire_wb(24, 0)
        wait_wb(23, 1)
        wait_wb(24, 0)

    out = gather_rows(qx, qy, table)
    return out.reshape(B, _KK, H)


# speedup vs baseline: 1.0576x; 1.0009x over previous
import functools

import jax
import jax.numpy as jnp
from jax import lax
from jax.experimental import pallas as pl
from jax.experimental.pallas import tpu as pltpu
from jax.experimental.pallas import tpu_sc as plsc

_W = 2
_K = 5
_KK = _K * _K


def kernel(grid_input, w, memory):
    B = grid_input.shape[0]
    N, M, H = memory.shape
    assert B == 4096 and N == 512 and M == 512 and H == 256
    table = memory.reshape(N * M, H)
    grid_adj = grid_input + (w - _W)
    qx = grid_adj[:, 0]
    qy = grid_adj[:, 1]

    R = B * _KK
    NC, NS, L = 2, 16, 16
    NW = NC * NS
    rows_per_w = R // NW
    CHUNK = 128
    n_chunks = rows_per_w // CHUNK

    mesh = plsc.VectorSubcoreMesh(core_axis_name="c", subcore_axis_name="s")

    @functools.partial(
        pl.kernel,
        mesh=mesh,
        compiler_params=pltpu.CompilerParams(use_tc_tiling_on_sc=True),
        out_type=jax.ShapeDtypeStruct((R, H), jnp.float32),
        scratch_types=[
            pltpu.VMEM((B,), jnp.int32),
            pltpu.VMEM((B,), jnp.int32),
            pltpu.VMEM((CHUNK,), jnp.int32),
            pltpu.VMEM((CHUNK,), jnp.int32),
            pltpu.VMEM((CHUNK, H), jnp.float32),
            pltpu.VMEM((CHUNK, H), jnp.float32),
            pltpu.SemaphoreType.DMA,
            pltpu.SemaphoreType.DMA,
            pltpu.SemaphoreType.DMA,
            pltpu.SemaphoreType.DMA,
        ],
    )
    def gather_rows(qx_hbm, qy_hbm, table_hbm, out_hbm,
                    qx_v, qy_v, idx0_v, idx1_v, rows0_v, rows1_v,
                    sg0, sg1, sw0, sw1):
        wid = lax.axis_index("s") * NC + lax.axis_index("c")
        base = wid * rows_per_w
        pltpu.sync_copy(qx_hbm, qx_v)
        pltpu.sync_copy(qy_hbm, qy_v)

        idx_v = (idx0_v, idx1_v)
        rows_v = (rows0_v, rows1_v)
        sg = (sg0, sg1)
        sw = (sw0, sw1)

        def compute_idx(c, dst):
            start = base + c * CHUNK
            q = lax.shift_right_logical(start, 12)
            i = lax.shift_right_logical(q * 13108, 16)
            j = q - i * _K
            b0 = jnp.bitwise_and(start, B - 1)
            for v in range(CHUNK // L):
                gx = qx_v[pl.ds(b0 + v * L, L)]
                gy = qy_v[pl.ds(b0 + v * L, L)]
                rx = jnp.bitwise_and(gx + (i - _W), N - 1)
                ry = jnp.bitwise_and(gy + (j - _W), M - 1)
                dst[pl.ds(v * L, L)] = lax.shift_left(rx, 9) + ry

        def fire_gather(c, p):
            pltpu.async_copy(table_hbm.at[idx_v[p]], rows_v[p], sg[p])

        def wait_gather(p):
            pltpu.make_async_copy(table_hbm.at[idx_v[p]], rows_v[p],
                                  sg[p]).wait()

        def fire_wb(c, p):
            pltpu.async_copy(rows_v[p],
                             out_hbm.at[pl.ds(base + c * CHUNK, CHUNK)], sw[p])

        def wait_wb(c, p):
            pltpu.make_async_copy(rows_v[p],
                                  out_hbm.at[pl.ds(base + c * CHUNK, CHUNK)],
                                  sw[p]).wait()

        compute_idx(0, idx_v[0])
        fire_gather(0, 0)
        compute_idx(1, idx_v[1])
        fire_gather(1, 1)

        def step(c, p):
            wait_gather(p)
            fire_wb(c, p)
            compute_idx(c + 2, idx_v[p])
            wait_wb(c, p)
            fire_gather(c + 2, p)

        def body(it, carry):
            c0 = it * 2
            step(c0, 0)
            step(c0 + 1, 1)
            return carry

        lax.fori_loop(0, 11, body, 0)
        step(22, 0)
        wait_gather(1)
        fire_wb(23, 1)
        wait_gather(0)
        fire_wb(24, 0)
        wait_wb(23, 1)
        wait_wb(24, 0)

    out = gather_rows(qx, qy, table)
    return out.reshape(B, _KK, H)
